# speedup vs baseline: 4.1036x; 4.1036x over previous
import functools

import jax
import jax.numpy as jnp
from jax import lax
from jax.experimental import pallas as pl
from jax.experimental.pallas import tpu as pltpu
from jax.experimental.pallas import tpu_sc as plsc

_NC = 2
_NS = 16
_NW = _NC * _NS

_HIDDEN = 128
_CHUNK = 128


def _table_body(emb_ref, w_ref, b_ref, out_ref):
    out_ref[...] = (
        lax.dot_general(
            emb_ref[...],
            w_ref[...],
            (((1,), (1,)), ((), ())),
            preferred_element_type=jnp.float32,
        )
        + b_ref[...]
    )


def _make_table(emb, W, b):
    return pl.pallas_call(
        _table_body,
        out_shape=jax.ShapeDtypeStruct((emb.shape[0], W.shape[0]), jnp.float32),
    )(emb, W, b.reshape(1, -1))


def _gather_body(n_chunks, table_hbm, idx_hbm, out_hbm, idx_v, rows_v, sem):
    wid = lax.axis_index("s") * _NC + lax.axis_index("c")
    pltpu.sync_copy(idx_hbm.at[pl.ds(wid * n_chunks, n_chunks)], idx_v)
    base = wid * n_chunks * _CHUNK

    def step(j, carry):
        pltpu.async_copy(table_hbm.at[idx_v.at[j]], rows_v, sem).wait()
        pltpu.sync_copy(rows_v, out_hbm.at[pl.ds(base + j * _CHUNK, _CHUNK)])
        return carry

    lax.fori_loop(0, n_chunks, step, 0)


def _gather(table, idx2):
    total = idx2.shape[0] * idx2.shape[1]
    n_chunks = idx2.shape[0] // _NW
    mesh = plsc.VectorSubcoreMesh(
        core_axis_name="c", subcore_axis_name="s", num_cores=_NC, num_subcores=_NS
    )
    return pl.kernel(
        functools.partial(_gather_body, n_chunks),
        out_type=jax.ShapeDtypeStruct((total, _HIDDEN), jnp.float32),
        mesh=mesh,
        scratch_types=[
            pltpu.VMEM((n_chunks, _CHUNK), jnp.int32),
            pltpu.VMEM((_CHUNK, _HIDDEN), jnp.float32),
            pltpu.SemaphoreType.DMA,
        ],
    )(table, idx2)


def kernel(x, emb, W, b):
    batch, hist = x.shape
    total = batch * hist
    idx2 = x.reshape(total // _CHUNK, _CHUNK).astype(jnp.int32)
    table = _make_table(emb, W, b)
    out = _gather(table, idx2)
    return out.reshape(batch, hist, _HIDDEN)


# speedup vs baseline: 4.1852x; 1.0199x over previous
import functools

import jax
import jax.numpy as jnp
from jax import lax
from jax.experimental import pallas as pl
from jax.experimental.pallas import tpu as pltpu
from jax.experimental.pallas import tpu_sc as plsc

_NC = 2
_NS = 16
_NW = _NC * _NS

_HIDDEN = 128
_CHUNK = 128


def _table_body(emb_ref, w_ref, b_ref, out_ref):
    out_ref[...] = (
        lax.dot_general(
            emb_ref[...],
            w_ref[...],
            (((1,), (1,)), ((), ())),
            preferred_element_type=jnp.float32,
        )
        + b_ref[...]
    )


def _make_table(emb, W, b):
    return pl.pallas_call(
        _table_body,
        out_shape=jax.ShapeDtypeStruct((emb.shape[0], W.shape[0]), jnp.float32),
    )(emb, W, b.reshape(1, -1))


_NB = 4


def _gather_body(n_chunks, table_hbm, idx_hbm, out_hbm, idx_v, rows_v, gsem, wsem):
    wid = lax.axis_index("s") * _NC + lax.axis_index("c")
    pltpu.sync_copy(idx_hbm.at[pl.ds(wid * n_chunks, n_chunks)], idx_v)
    base = wid * n_chunks * _CHUNK

    def g_start(j, b):
        pltpu.async_copy(table_hbm.at[idx_v.at[j]], rows_v.at[b], gsem.at[b])

    def g_wait(j, b):
        pltpu.make_async_copy(
            table_hbm.at[idx_v.at[j]], rows_v.at[b], gsem.at[b]
        ).wait()

    def w_start(j, b):
        pltpu.async_copy(
            rows_v.at[b], out_hbm.at[pl.ds(base + j * _CHUNK, _CHUNK)], wsem.at[b]
        )

    def w_wait(j, b):
        pltpu.make_async_copy(
            rows_v.at[b], out_hbm.at[pl.ds(base + j * _CHUNK, _CHUNK)], wsem.at[b]
        ).wait()

    g_start(0, 0)
    for j in range(_NB):
        g_wait(j, j)
        w_start(j, j)
        if j < _NB - 1:
            g_start(j + 1, j + 1)
    w_wait(0, 0)
    g_start(_NB, 0)

    def grp(j2, carry):
        for u in range(_NB):
            j = j2 * _NB + u
            bn = (u + 1) % _NB
            g_wait(j, u)
            w_start(j, u)
            w_wait(j - (_NB - 1), bn)
            g_start(j + 1, bn)
        return carry

    lax.fori_loop(1, n_chunks // _NB - 1, grp, 0)

    for u in range(_NB):
        j = n_chunks - _NB + u
        bn = (u + 1) % _NB
        g_wait(j, u)
        w_start(j, u)
        if u < _NB - 1:
            w_wait(j - (_NB - 1), bn)
            g_start(j + 1, bn)
    for u in range(_NB):
        w_wait(n_chunks - _NB + u, u)


def _gather(table, idx2):
    total = idx2.shape[0] * idx2.shape[1]
    n_chunks = idx2.shape[0] // _NW
    mesh = plsc.VectorSubcoreMesh(
        core_axis_name="c", subcore_axis_name="s", num_cores=_NC, num_subcores=_NS
    )
    return pl.kernel(
        functools.partial(_gather_body, n_chunks),
        out_type=jax.ShapeDtypeStruct((total, _HIDDEN), jnp.float32),
        mesh=mesh,
        scratch_types=[
            pltpu.VMEM((n_chunks, _CHUNK), jnp.int32),
            pltpu.VMEM((_NB, _CHUNK, _HIDDEN), jnp.float32),
            pltpu.SemaphoreType.DMA((_NB,)),
            pltpu.SemaphoreType.DMA((_NB,)),
        ],
    )(table, idx2)


def kernel(x, emb, W, b):
    batch, hist = x.shape
    total = batch * hist
    idx2 = x.reshape(total // _CHUNK, _CHUNK).astype(jnp.int32)
    table = _make_table(emb, W, b)
    out = _gather(table, idx2)
    return out.reshape(batch, hist, _HIDDEN)


# speedup vs baseline: 16.6320x; 3.9740x over previous
import functools

import jax
import jax.numpy as jnp
from jax import lax
from jax.experimental import pallas as pl
from jax.experimental.pallas import tpu as pltpu
from jax.experimental.pallas import tpu_sc as plsc

_NC = 2
_NS = 16
_NW = _NC * _NS

_HIDDEN = 128
_CHUNK = 128


def _table_body(emb_ref, w_ref, b_ref, out_ref):
    out_ref[...] = (
        lax.dot_general(
            emb_ref[...],
            w_ref[...],
            (((1,), (1,)), ((), ())),
            preferred_element_type=jnp.float32,
        )
        + b_ref[...]
    )


def _make_table(emb, W, b):
    return pl.pallas_call(
        _table_body,
        out_shape=jax.ShapeDtypeStruct((emb.shape[0], W.shape[0]), jnp.float32),
    )(emb, W, b.reshape(1, -1))


_NB = 4


def _gather_body(n_chunks, table_hbm, idx_hbm, out_hbm, table_v, idx_v, rows_v, gsem, wsem):
    wid = lax.axis_index("s") * _NC + lax.axis_index("c")
    pltpu.sync_copy(table_hbm, table_v)
    pltpu.sync_copy(idx_hbm.at[pl.ds(wid * n_chunks, n_chunks)], idx_v)
    base = wid * n_chunks * _CHUNK

    def g_start(j, b):
        pltpu.async_copy(table_v.at[idx_v.at[j]], rows_v.at[b], gsem.at[b])

    def g_wait(j, b):
        pltpu.make_async_copy(
            table_v.at[idx_v.at[j]], rows_v.at[b], gsem.at[b]
        ).wait()

    def w_start(j, b):
        pltpu.async_copy(
            rows_v.at[b], out_hbm.at[pl.ds(base + j * _CHUNK, _CHUNK)], wsem.at[b]
        )

    def w_wait(j, b):
        pltpu.make_async_copy(
            rows_v.at[b], out_hbm.at[pl.ds(base + j * _CHUNK, _CHUNK)], wsem.at[b]
        ).wait()

    g_start(0, 0)
    for j in range(_NB):
        g_wait(j, j)
        w_start(j, j)
        if j < _NB - 1:
            g_start(j + 1, j + 1)
    w_wait(0, 0)
    g_start(_NB, 0)

    def grp(j2, carry):
        for u in range(_NB):
            j = j2 * _NB + u
            bn = (u + 1) % _NB
            g_wait(j, u)
            w_start(j, u)
            w_wait(j - (_NB - 1), bn)
            g_start(j + 1, bn)
        return carry

    lax.fori_loop(1, n_chunks // _NB - 1, grp, 0)

    for u in range(_NB):
        j = n_chunks - _NB + u
        bn = (u + 1) % _NB
        g_wait(j, u)
        w_start(j, u)
        if u < _NB - 1:
            w_wait(j - (_NB - 1), bn)
            g_start(j + 1, bn)
    for u in range(_NB):
        w_wait(n_chunks - _NB + u, u)


def _gather(table, idx2):
    total = idx2.shape[0] * idx2.shape[1]
    n_chunks = idx2.shape[0] // _NW
    mesh = plsc.VectorSubcoreMesh(
        core_axis_name="c", subcore_axis_name="s", num_cores=_NC, num_subcores=_NS
    )
    return pl.kernel(
        functools.partial(_gather_body, n_chunks),
        out_type=jax.ShapeDtypeStruct((total, _HIDDEN), jnp.float32),
        mesh=mesh,
        scratch_types=[
            pltpu.VMEM_SHARED((_HIDDEN, _HIDDEN), jnp.float32),
            pltpu.VMEM((n_chunks, _CHUNK), jnp.int32),
            pltpu.VMEM((_NB, _CHUNK, _HIDDEN), jnp.float32),
            pltpu.SemaphoreType.DMA((_NB,)),
            pltpu.SemaphoreType.DMA((_NB,)),
        ],
    )(table, idx2)


def kernel(x, emb, W, b):
    batch, hist = x.shape
    total = batch * hist
    idx2 = x.reshape(total // _CHUNK, _CHUNK).astype(jnp.int32)
    table = _make_table(emb, W, b)
    out = _gather(table, idx2)
    return out.reshape(batch, hist, _HIDDEN)


# speedup vs baseline: 17.2804x; 1.0390x over previous
import functools

import jax
import jax.numpy as jnp
from jax import lax
from jax.experimental import pallas as pl
from jax.experimental.pallas import tpu as pltpu
from jax.experimental.pallas import tpu_sc as plsc

_NC = 2
_NS = 16
_NW = _NC * _NS

_HIDDEN = 128
_IW = 128
_K = 2
_CHUNK = _K * _IW
_NB = 3


def _table_body(emb_ref, w_ref, b_ref, out_ref):
    out_ref[...] = (
        lax.dot_general(
            emb_ref[...],
            w_ref[...],
            (((1,), (1,)), ((), ())),
            preferred_element_type=jnp.float32,
        )
        + b_ref[...]
    )


def _make_table(emb, W, b):
    return pl.pallas_call(
        _table_body,
        out_shape=jax.ShapeDtypeStruct((emb.shape[0], W.shape[0]), jnp.float32),
    )(emb, W, b.reshape(1, -1))


def _gather_body(n_big, table_hbm, idx_hbm, out_hbm, table_s, idx_v, rows_v, gsem, wsem):
    wid = lax.axis_index("s") * _NC + lax.axis_index("c")
    n_rows = n_big * _K
    pltpu.sync_copy(table_hbm, table_s)
    pltpu.sync_copy(idx_hbm.at[pl.ds(wid * n_rows, n_rows)], idx_v)
    base = wid * n_rows

    def g_start(j, b):
        for k in range(_K):
            pltpu.async_copy(
                table_s.at[idx_v.at[j * _K + k]], rows_v.at[b, k], gsem.at[b]
            )

    def g_wait(j, b):
        for k in range(_K):
            pltpu.make_async_copy(
                table_s.at[idx_v.at[j * _K + k]], rows_v.at[b, k], gsem.at[b]
            ).wait()

    def w_start(j, b):
        pltpu.async_copy(
            rows_v.at[b], out_hbm.at[pl.ds(base + j * _K, _K)], wsem.at[b]
        )

    def w_wait(j, b):
        pltpu.make_async_copy(
            rows_v.at[b], out_hbm.at[pl.ds(base + j * _K, _K)], wsem.at[b]
        ).wait()

    g_start(0, 0)
    for j in range(_NB):
        g_wait(j, j)
        w_start(j, j)
        if j < _NB - 1:
            g_start(j + 1, j + 1)
    w_wait(0, 0)
    g_start(_NB, 0)

    def grp(j2, carry):
        for u in range(_NB):
            j = j2 * _NB + u
            bn = (u + 1) % _NB
            g_wait(j, u)
            w_start(j, u)
            w_wait(j - (_NB - 1), bn)
            g_start(j + 1, bn)
        return carry

    n_grp = (n_big - 1) // _NB
    lax.fori_loop(1, n_grp, grp, 0)

    for j in range(n_grp * _NB, n_big):
        u = j % _NB
        bn = (u + 1) % _NB
        g_wait(j, u)
        w_start(j, u)
        if j < n_big - 1:
            w_wait(j - (_NB - 1), bn)
            g_start(j + 1, bn)
    for j in range(n_big - _NB, n_big):
        w_wait(j, j % _NB)


def _gather(table, idx2):
    total_rows = idx2.shape[0]
    n_big = total_rows // (_NW * _K)
    mesh = plsc.VectorSubcoreMesh(
        core_axis_name="c", subcore_axis_name="s", num_cores=_NC, num_subcores=_NS
    )
    return pl.kernel(
        functools.partial(_gather_body, n_big),
        out_type=jax.ShapeDtypeStruct((total_rows, _IW, _HIDDEN), jnp.float32),
        mesh=mesh,
        scratch_types=[
            pltpu.VMEM_SHARED((_HIDDEN, _HIDDEN), jnp.float32),
            pltpu.VMEM((n_big * _K, _IW), jnp.int32),
            pltpu.VMEM((_NB, _K, _IW, _HIDDEN), jnp.float32),
            pltpu.SemaphoreType.DMA((_NB,)),
            pltpu.SemaphoreType.DMA((_NB,)),
        ],
    )(table, idx2)


def kernel(x, emb, W, b):
    batch, hist = x.shape
    total = batch * hist
    idx2 = x.reshape(total // _IW, _IW).astype(jnp.int32)
    table = _make_table(emb, W, b)
    out = _gather(table, idx2)
    return out.reshape(batch, hist, _HIDDEN)


# speedup vs baseline: 17.3376x; 1.0033x over previous
import functools

import jax
import jax.numpy as jnp
from jax import lax
from jax.experimental import pallas as pl
from jax.experimental.pallas import tpu as pltpu
from jax.experimental.pallas import tpu_sc as plsc

_NC = 2
_NS = 16
_NW = _NC * _NS

_HIDDEN = 128
_IW = 128
_K = 2
_CHUNK = _K * _IW
_NB = 3


def _table_body(emb_ref, w_ref, b_ref, out_ref):
    out_ref[...] = (
        lax.dot_general(
            emb_ref[...],
            w_ref[...],
            (((1,), (1,)), ((), ())),
            preferred_element_type=jnp.float32,
        )
        + b_ref[...]
    )


def _make_table(emb, W, b):
    return pl.pallas_call(
        _table_body,
        out_shape=jax.ShapeDtypeStruct((emb.shape[0], W.shape[0]), jnp.float32),
    )(emb, W, b.reshape(1, -1))


def _gather_body(n_big, table_hbm, idx_hbm, out_hbm, table_s, idx_v, rows_v, gsem, wsem):
    sid = lax.axis_index("s")
    wid = sid * _NC + lax.axis_index("c")
    n_rows = n_big * _K
    @pl.when(sid == 0)
    def _stage_table():
        pltpu.sync_copy(table_hbm, table_s)

    pltpu.sync_copy(idx_hbm.at[pl.ds(wid * n_rows, n_rows)], idx_v)
    plsc.subcore_barrier()
    base = wid * n_rows

    def g_start(j, b):
        for k in range(_K):
            pltpu.async_copy(
                table_s.at[idx_v.at[j * _K + k]], rows_v.at[b, k], gsem.at[b]
            )

    def g_wait(j, b):
        for k in range(_K):
            pltpu.make_async_copy(
                table_s.at[idx_v.at[j * _K + k]], rows_v.at[b, k], gsem.at[b]
            ).wait()

    def w_start(j, b):
        pltpu.async_copy(
            rows_v.at[b], out_hbm.at[pl.ds(base + j * _K, _K)], wsem.at[b]
        )

    def w_wait(j, b):
        pltpu.make_async_copy(
            rows_v.at[b], out_hbm.at[pl.ds(base + j * _K, _K)], wsem.at[b]
        ).wait()

    g_start(0, 0)
    for j in range(_NB):
        g_wait(j, j)
        w_start(j, j)
        if j < _NB - 1:
            g_start(j + 1, j + 1)
    w_wait(0, 0)
    g_start(_NB, 0)

    def grp(j2, carry):
        for u in range(_NB):
            j = j2 * _NB + u
            bn = (u + 1) % _NB
            g_wait(j, u)
            w_start(j, u)
            w_wait(j - (_NB - 1), bn)
            g_start(j + 1, bn)
        return carry

    n_grp = (n_big - 1) // _NB
    lax.fori_loop(1, n_grp, grp, 0)

    for j in range(n_grp * _NB, n_big):
        u = j % _NB
        bn = (u + 1) % _NB
        g_wait(j, u)
        w_start(j, u)
        if j < n_big - 1:
            w_wait(j - (_NB - 1), bn)
            g_start(j + 1, bn)
    for j in range(n_big - _NB, n_big):
        w_wait(j, j % _NB)


def _gather(table, idx2):
    total_rows = idx2.shape[0]
    n_big = total_rows // (_NW * _K)
    mesh = plsc.VectorSubcoreMesh(
        core_axis_name="c", subcore_axis_name="s", num_cores=_NC, num_subcores=_NS
    )
    return pl.kernel(
        functools.partial(_gather_body, n_big),
        out_type=jax.ShapeDtypeStruct((total_rows, _IW, _HIDDEN), jnp.float32),
        mesh=mesh,
        scratch_types=[
            pltpu.VMEM_SHARED((_HIDDEN, _HIDDEN), jnp.float32),
            pltpu.VMEM((n_big * _K, _IW), jnp.int32),
            pltpu.VMEM((_NB, _K, _IW, _HIDDEN), jnp.float32),
            pltpu.SemaphoreType.DMA((_NB,)),
            pltpu.SemaphoreType.DMA((_NB,)),
        ],
    )(table, idx2)


def kernel(x, emb, W, b):
    batch, hist = x.shape
    total = batch * hist
    idx2 = x.reshape(total // _IW, _IW).astype(jnp.int32)
    table = _make_table(emb, W, b)
    out = _gather(table, idx2)
    return out.reshape(batch, hist, _HIDDEN)
